# Initial kernel scaffold; baseline (speedup 1.0000x reference)
import jax
import jax.numpy as jnp
from jax.experimental import pallas as pl


def kernel(input_time_matrix, emb_table, W, b):
    raise NotImplementedError("write your pallas kernel here")


# speedup vs baseline: 80.3094x; 80.3094x over previous
import functools

import jax
import jax.numpy as jnp
from jax import lax
from jax.experimental import pallas as pl
from jax.experimental.pallas import tpu as pltpu
from jax.experimental.pallas import tpu_sc as plsc

B = 1024
L = 50
LL = L * L
H = 8
V = 1025
LANES = 16

NC = 2
NS = 16
NW = NC * NS
BPW = B // NW
CB = 2
NCHUNK = BPW // CB
CHUNK_IDX = CB * LL
CHUNK_OUT = CB * H * LL
NVEC = (CHUNK_IDX + LANES - 1) // LANES
IDX_PAD = NVEC * LANES


def _ptable_body(emb_ref, w_ref, b_ref, out_ref):
    out_ref[...] = (
        jax.lax.dot_general(
            w_ref[...], emb_ref[...], (((0,), (1,)), ((), ())),
            preferred_element_type=jnp.float32,
        )
        + b_ref[...]
    )


def _make_ptable(emb_table, w, bias):
    return pl.pallas_call(
        _ptable_body,
        out_shape=jax.ShapeDtypeStruct((H, V), jnp.float32),
    )(emb_table, w, bias.reshape(H, 1))


def _sc_body(ptable_hbm, idx_hbm, out_hbm,
             table_v, idx_v0, idx_v1, out_v0, out_v1,
             sem_i0, sem_i1, sem_o0, sem_o1):
    wid = lax.axis_index("s") * NC + lax.axis_index("c")
    idx_bufs = (idx_v0, idx_v1)
    out_bufs = (out_v0, out_v1)
    isems = (sem_i0, sem_i1)
    osems = (sem_o0, sem_o1)

    iota = lax.iota(jnp.int32, LANES)

    zeros = jnp.zeros((LANES,), jnp.int32)
    idx_v0[pl.ds(IDX_PAD - LANES, LANES)] = zeros
    idx_v1[pl.ds(IDX_PAD - LANES, LANES)] = zeros

    pltpu.sync_copy(ptable_hbm, table_v)

    def idx_src(c):
        off = wid * (BPW * LL) + c * CHUNK_IDX
        return idx_hbm.at[pl.ds(off, CHUNK_IDX)]

    def out_dst(c):
        off = wid * (BPW * H * LL) + c * CHUNK_OUT
        return out_hbm.at[pl.ds(off, CHUNK_OUT)]

    def start_idx(c, buf):
        pltpu.make_async_copy(
            idx_src(c), idx_bufs[buf].at[pl.ds(0, CHUNK_IDX)], isems[buf]
        ).start()

    def wait_idx(c, buf):
        pltpu.make_async_copy(
            idx_src(c), idx_bufs[buf].at[pl.ds(0, CHUNK_IDX)], isems[buf]
        ).wait()

    def start_out(c, buf):
        pltpu.make_async_copy(out_bufs[buf], out_dst(c), osems[buf]).start()

    def wait_out(c, buf):
        pltpu.make_async_copy(out_bufs[buf], out_dst(c), osems[buf]).wait()

    def compute(idx_ref, out_ref):
        def body(i, carry):
            base = i * LANES
            p = base + iota
            idxv = idx_ref[pl.ds(base, LANES)]
            q = (p >= LL).astype(jnp.int32) + (p >= CHUNK_IDX).astype(jnp.int32)
            pos0 = p + q * ((H - 1) * LL)
            mask = p < CHUNK_IDX
            for h in range(H):
                vals = plsc.load_gather(table_v, [idxv + h * V])
                plsc.store_scatter(out_ref, [pos0 + h * LL], vals, mask=mask)
            return carry

        lax.fori_loop(0, NVEC, body, 0)

    start_idx(0, 0)
    start_idx(1, 1)
    for c in range(NCHUNK):
        buf = c % 2
        wait_idx(c, buf)
        if c >= 2:
            wait_out(c - 2, buf)
        compute(idx_bufs[buf], out_bufs[buf])
        start_out(c, buf)
        if c + 2 < NCHUNK:
            start_idx(c + 2, buf)
    wait_out(NCHUNK - 2, 0)
    wait_out(NCHUNK - 1, 1)


_sc_gather = pl.kernel(
    _sc_body,
    out_type=jax.ShapeDtypeStruct((B * H * LL,), jnp.float32),
    mesh=plsc.VectorSubcoreMesh(core_axis_name="c", subcore_axis_name="s"),
    compiler_params=pltpu.CompilerParams(needs_layout_passes=False),
    scratch_types=[
        pltpu.VMEM((H * V,), jnp.float32),
        pltpu.VMEM((IDX_PAD,), jnp.int32),
        pltpu.VMEM((IDX_PAD,), jnp.int32),
        pltpu.VMEM((CHUNK_OUT,), jnp.float32),
        pltpu.VMEM((CHUNK_OUT,), jnp.float32),
        pltpu.SemaphoreType.DMA,
        pltpu.SemaphoreType.DMA,
        pltpu.SemaphoreType.DMA,
        pltpu.SemaphoreType.DMA,
    ],
)


def kernel(input_time_matrix, emb_table, W, b):
    idx_flat = input_time_matrix.reshape(-1)
    if idx_flat.dtype != jnp.int32:
        idx_flat = idx_flat.astype(jnp.int32)
    ptable = _make_ptable(emb_table, W, b).reshape(-1)
    out_flat = _sc_gather(ptable, idx_flat)
    return out_flat.reshape(B, H, L, L)


# speedup vs baseline: 208.6899x; 2.5986x over previous
import jax
import jax.numpy as jnp
from jax import lax
from jax.experimental import pallas as pl
from jax.experimental.pallas import tpu as pltpu
from jax.experimental.pallas import tpu_sc as plsc

B = 1024
L = 50
P = L * L
H = 8
V = 1025
LANES = 16

NC = 2
NS = 16
NW = NC * NS
CP = 4
NCHUNK = P // CP
CPW = -(-NCHUNK // NW)
CHUNK_IDX = CP * B
CHUNK_OUT = CP * H * B
VPB = B // LANES


def _ptable_body(emb_ref, w_ref, b_ref, out_ref):
    out_ref[...] = (
        jax.lax.dot_general(
            w_ref[...], emb_ref[...], (((0,), (1,)), ((), ())),
            preferred_element_type=jnp.float32,
        )
        + b_ref[...]
    )


def _make_ptable(emb_table, w, bias):
    return pl.pallas_call(
        _ptable_body,
        out_shape=jax.ShapeDtypeStruct((H, V), jnp.float32),
    )(emb_table, w, bias.reshape(H, 1))


def _sc_body(ptable_hbm, idx_hbm, out_hbm,
             table_v, idx_v0, idx_v1, out_v0, out_v1,
             sem_i0, sem_i1, sem_o0, sem_o1):
    wid = lax.axis_index("s") * NC + lax.axis_index("c")
    idx_bufs = (idx_v0, idx_v1)
    out_bufs = (out_v0, out_v1)
    isems = (sem_i0, sem_i1)
    osems = (sem_o0, sem_o1)

    pltpu.sync_copy(ptable_hbm, table_v)

    g0 = wid * CPW

    def idx_src(g):
        return idx_hbm.at[pl.ds(g * CHUNK_IDX, CHUNK_IDX)]

    def out_dst(g):
        return out_hbm.at[pl.ds(g * CHUNK_OUT, CHUNK_OUT)]

    def start_idx(g, buf):
        pltpu.make_async_copy(idx_src(g), idx_bufs[buf], isems[buf]).start()

    def wait_idx(g, buf):
        pltpu.make_async_copy(idx_src(g), idx_bufs[buf], isems[buf]).wait()

    def start_out(g, buf):
        pltpu.make_async_copy(out_bufs[buf], out_dst(g), osems[buf]).start()

    def wait_out(g, buf):
        pltpu.make_async_copy(out_bufs[buf], out_dst(g), osems[buf]).wait()

    def compute(idx_ref, out_ref):
        for po in range(CP):
            def body(v, carry):
                k = v // 8
                r = (v % 8) * LANES
                idxv = idx_ref[pl.ds(po * B + v * LANES, LANES)]
                base = po * (H * B) + k * (H * 128) + r
                for h in range(H):
                    vals = plsc.load_gather(table_v, [idxv + h * V])
                    out_ref[pl.ds(base + h * 128, LANES)] = vals
                return carry

            lax.fori_loop(0, VPB, body, 0)

    def guarded(g, fn, *args):
        @pl.when(g < NCHUNK)
        def _():
            fn(*args)

    guarded(g0, start_idx, g0, 0)
    guarded(g0 + 1, start_idx, g0 + 1, 1)
    for c in range(CPW):
        g = g0 + c
        buf = c % 2

        @pl.when(g < NCHUNK)
        def _(g=g, buf=buf, c=c):
            wait_idx(g, buf)
            if c >= 2:
                wait_out(g - 2, buf)
            compute(idx_bufs[buf], out_bufs[buf])
            start_out(g, buf)
            if c + 2 < CPW:
                guarded(g + 2, start_idx, g + 2, buf)

    guarded(g0 + CPW - 2, wait_out, g0 + CPW - 2, (CPW - 2) % 2)
    guarded(g0 + CPW - 1, wait_out, g0 + CPW - 1, (CPW - 1) % 2)


_sc_gather = pl.kernel(
    _sc_body,
    out_type=jax.ShapeDtypeStruct((P * H * B,), jnp.float32),
    mesh=plsc.VectorSubcoreMesh(core_axis_name="c", subcore_axis_name="s"),
    compiler_params=pltpu.CompilerParams(needs_layout_passes=False),
    scratch_types=[
        pltpu.VMEM((H * V,), jnp.float32),
        pltpu.VMEM((CHUNK_IDX,), jnp.int32),
        pltpu.VMEM((CHUNK_IDX,), jnp.int32),
        pltpu.VMEM((CHUNK_OUT,), jnp.float32),
        pltpu.VMEM((CHUNK_OUT,), jnp.float32),
        pltpu.SemaphoreType.DMA,
        pltpu.SemaphoreType.DMA,
        pltpu.SemaphoreType.DMA,
        pltpu.SemaphoreType.DMA,
    ],
)


def kernel(input_time_matrix, emb_table, W, b):
    idx_t = jnp.transpose(input_time_matrix, (1, 2, 0)).reshape(-1)
    if idx_t.dtype != jnp.int32:
        idx_t = idx_t.astype(jnp.int32)
    ptable = _make_ptable(emb_table, W, b).reshape(-1)
    out_flat = _sc_gather(ptable, idx_t)
    out = out_flat.reshape(L, L, B // 128, H, 128)
    return jnp.transpose(out, (2, 4, 3, 0, 1)).reshape(B, H, L, L)


# speedup vs baseline: 642.2104x; 3.0773x over previous
import jax
import jax.numpy as jnp
from jax import lax
from jax.experimental import pallas as pl
from jax.experimental.pallas import tpu as pltpu
from jax.experimental.pallas import tpu_sc as plsc

B = 1024
L = 50
P = L * L
H = 8
V = 1025
LANES = 16

NC = 2
NS = 16
NW = NC * NS
CP = 4
NCHUNK = P // CP
CPW = -(-NCHUNK // NW)
CHUNK_IDX = CP * B
CHUNK_OUT = CP * H * B
VPB = B // LANES


def _ptable_body(emb_ref, w_ref, b_ref, out_ref):
    out_ref[...] = (
        jax.lax.dot_general(
            w_ref[...], emb_ref[...], (((0,), (1,)), ((), ())),
            preferred_element_type=jnp.float32,
        )
        + b_ref[...]
    )


def _make_ptable(emb_table, w, bias):
    return pl.pallas_call(
        _ptable_body,
        out_shape=jax.ShapeDtypeStruct((H, V), jnp.float32),
    )(emb_table, w, bias.reshape(H, 1))


def _sc_body(ptable_hbm, idx_hbm, out_hbm,
             table_v, idx_v0, idx_v1, out_v0, out_v1,
             sem_i0, sem_i1, sem_o0, sem_o1):
    wid = lax.axis_index("s") * NC + lax.axis_index("c")
    idx_bufs = (idx_v0, idx_v1)
    out_bufs = (out_v0, out_v1)
    isems = (sem_i0, sem_i1)
    osems = (sem_o0, sem_o1)

    pltpu.sync_copy(ptable_hbm, table_v)

    g0 = wid * CPW

    def idx_src(g):
        return idx_hbm.at[pl.ds(g * CHUNK_IDX, CHUNK_IDX)]

    def out_dst(g):
        return out_hbm.at[pl.ds(g * CHUNK_OUT, CHUNK_OUT)]

    def start_idx(g, buf):
        pltpu.make_async_copy(idx_src(g), idx_bufs[buf], isems[buf]).start()

    def wait_idx(g, buf):
        pltpu.make_async_copy(idx_src(g), idx_bufs[buf], isems[buf]).wait()

    def start_out(g, buf):
        pltpu.make_async_copy(out_bufs[buf], out_dst(g), osems[buf]).start()

    def wait_out(g, buf):
        pltpu.make_async_copy(out_bufs[buf], out_dst(g), osems[buf]).wait()

    def compute(idx_ref, out_ref):
        @plsc.parallel_loop(0, CP * VPB, unroll=4)
        def _(v):
            idxv = idx_ref[pl.ds(v * LANES, LANES)]
            base = (v // 8) * (H * 128) + (v % 8) * LANES
            for h in range(H):
                vals = plsc.load_gather(table_v, [idxv + h * V])
                out_ref[pl.ds(base + h * 128, LANES)] = vals

    def guarded(g, fn, *args):
        @pl.when(g < NCHUNK)
        def _():
            fn(*args)

    guarded(g0, start_idx, g0, 0)
    guarded(g0 + 1, start_idx, g0 + 1, 1)
    for c in range(CPW):
        g = g0 + c
        buf = c % 2

        @pl.when(g < NCHUNK)
        def _(g=g, buf=buf, c=c):
            wait_idx(g, buf)
            if c >= 2:
                wait_out(g - 2, buf)
            compute(idx_bufs[buf], out_bufs[buf])
            start_out(g, buf)
            if c + 2 < CPW:
                guarded(g + 2, start_idx, g + 2, buf)

    guarded(g0 + CPW - 2, wait_out, g0 + CPW - 2, (CPW - 2) % 2)
    guarded(g0 + CPW - 1, wait_out, g0 + CPW - 1, (CPW - 1) % 2)


_sc_gather = pl.kernel(
    _sc_body,
    out_type=jax.ShapeDtypeStruct((P * H * B,), jnp.float32),
    mesh=plsc.VectorSubcoreMesh(core_axis_name="c", subcore_axis_name="s"),
    compiler_params=pltpu.CompilerParams(needs_layout_passes=False),
    scratch_types=[
        pltpu.VMEM((H * V,), jnp.float32),
        pltpu.VMEM((CHUNK_IDX,), jnp.int32),
        pltpu.VMEM((CHUNK_IDX,), jnp.int32),
        pltpu.VMEM((CHUNK_OUT,), jnp.float32),
        pltpu.VMEM((CHUNK_OUT,), jnp.float32),
        pltpu.SemaphoreType.DMA,
        pltpu.SemaphoreType.DMA,
        pltpu.SemaphoreType.DMA,
        pltpu.SemaphoreType.DMA,
    ],
)


def kernel(input_time_matrix, emb_table, W, b):
    idx_t = jnp.transpose(input_time_matrix, (1, 2, 0)).reshape(-1)
    if idx_t.dtype != jnp.int32:
        idx_t = idx_t.astype(jnp.int32)
    ptable = _make_ptable(emb_table, W, b).reshape(-1)
    out_flat = _sc_gather(ptable, idx_t)
    out = out_flat.reshape(L, L, B // 128, H, 128)
    return jnp.transpose(out, (2, 4, 3, 0, 1)).reshape(B, H, L, L)


# speedup vs baseline: 648.8390x; 1.0103x over previous
import jax
import jax.numpy as jnp
from jax import lax
from jax.experimental import pallas as pl
from jax.experimental.pallas import tpu as pltpu
from jax.experimental.pallas import tpu_sc as plsc

B = 1024
L = 50
P = L * L
H = 8
V = 1025
LANES = 16

NC = 2
NS = 16
NW = NC * NS
CP = 4
NCHUNK = P // CP
CPW = -(-NCHUNK // NW)
CHUNK_IDX = CP * B
CHUNK_OUT = CP * H * B
VPB = B // LANES


def _ptable_body(emb_ref, w_ref, b_ref, out_ref):
    out_ref[...] = (
        jax.lax.dot_general(
            w_ref[...], emb_ref[...], (((0,), (1,)), ((), ())),
            preferred_element_type=jnp.float32,
        )
        + b_ref[...]
    )


def _make_ptable(emb_table, w, bias):
    return pl.pallas_call(
        _ptable_body,
        out_shape=jax.ShapeDtypeStruct((H, V), jnp.float32),
    )(emb_table, w, bias.reshape(H, 1))


def _sc_body(ptable_hbm, idx_hbm, out_hbm,
             table_v, idx_v0, idx_v1, out_v0, out_v1,
             sem_i0, sem_i1, sem_o0, sem_o1):
    wid = lax.axis_index("s") * NC + lax.axis_index("c")
    idx_bufs = (idx_v0, idx_v1)
    out_bufs = (out_v0, out_v1)
    isems = (sem_i0, sem_i1)
    osems = (sem_o0, sem_o1)

    pltpu.sync_copy(ptable_hbm, table_v)

    g0 = wid * CPW

    def idx_src(g):
        return idx_hbm.at[pl.ds(g * CHUNK_IDX, CHUNK_IDX)]

    def out_dst(g):
        return out_hbm.at[pl.ds(g * CHUNK_OUT, CHUNK_OUT)]

    def start_idx(g, buf):
        pltpu.make_async_copy(idx_src(g), idx_bufs[buf], isems[buf]).start()

    def wait_idx(g, buf):
        pltpu.make_async_copy(idx_src(g), idx_bufs[buf], isems[buf]).wait()

    def start_out(g, buf):
        pltpu.make_async_copy(out_bufs[buf], out_dst(g), osems[buf]).start()

    def wait_out(g, buf):
        pltpu.make_async_copy(out_bufs[buf], out_dst(g), osems[buf]).wait()

    def compute(idx_ref, out_ref):
        @plsc.parallel_loop(0, CP * VPB, unroll=8)
        def _(v):
            idxv = idx_ref[pl.ds(v * LANES, LANES)]
            base = (v // 8) * (H * 128) + (v % 8) * LANES
            for h in range(H):
                vals = plsc.load_gather(table_v, [idxv + h * V])
                out_ref[pl.ds(base + h * 128, LANES)] = vals

    def guarded(g, fn, *args):
        @pl.when(g < NCHUNK)
        def _():
            fn(*args)

    guarded(g0, start_idx, g0, 0)
    guarded(g0 + 1, start_idx, g0 + 1, 1)
    for c in range(CPW):
        g = g0 + c
        buf = c % 2

        @pl.when(g < NCHUNK)
        def _(g=g, buf=buf, c=c):
            wait_idx(g, buf)
            if c >= 2:
                wait_out(g - 2, buf)
            compute(idx_bufs[buf], out_bufs[buf])
            start_out(g, buf)
            if c + 2 < CPW:
                guarded(g + 2, start_idx, g + 2, buf)

    guarded(g0 + CPW - 2, wait_out, g0 + CPW - 2, (CPW - 2) % 2)
    guarded(g0 + CPW - 1, wait_out, g0 + CPW - 1, (CPW - 1) % 2)


_sc_gather = pl.kernel(
    _sc_body,
    out_type=jax.ShapeDtypeStruct((P * H * B,), jnp.float32),
    mesh=plsc.VectorSubcoreMesh(core_axis_name="c", subcore_axis_name="s"),
    compiler_params=pltpu.CompilerParams(needs_layout_passes=False),
    scratch_types=[
        pltpu.VMEM((H * V,), jnp.float32),
        pltpu.VMEM((CHUNK_IDX,), jnp.int32),
        pltpu.VMEM((CHUNK_IDX,), jnp.int32),
        pltpu.VMEM((CHUNK_OUT,), jnp.float32),
        pltpu.VMEM((CHUNK_OUT,), jnp.float32),
        pltpu.SemaphoreType.DMA,
        pltpu.SemaphoreType.DMA,
        pltpu.SemaphoreType.DMA,
        pltpu.SemaphoreType.DMA,
    ],
)


def kernel(input_time_matrix, emb_table, W, b):
    idx_t = jnp.transpose(input_time_matrix, (1, 2, 0)).reshape(-1)
    if idx_t.dtype != jnp.int32:
        idx_t = idx_t.astype(jnp.int32)
    ptable = _make_ptable(emb_table, W, b).reshape(-1)
    out_flat = _sc_gather(ptable, idx_t)
    out = out_flat.reshape(L, L, B // 128, H, 128)
    return jnp.transpose(out, (2, 4, 3, 0, 1)).reshape(B, H, L, L)
